# Initial kernel scaffold; baseline (speedup 1.0000x reference)
import functools

import jax
import jax.numpy as jnp
from jax import lax
from jax.experimental import pallas as pl
from jax.experimental.pallas import tpu as pltpu
from jax.experimental.pallas import tpu_sc as plsc

_NC = 2
_NS = 16
_CH = 128


def _sc_gather(table2d, idx2d, bt, d):
    nw = _NC * _NS
    per_w = bt // nw
    n_ch = per_w // _CH
    mesh = plsc.VectorSubcoreMesh(core_axis_name="c", subcore_axis_name="s")

    @functools.partial(
        pl.kernel,
        mesh=mesh,
        out_type=jax.ShapeDtypeStruct((bt, d), jnp.float32),
        scratch_types=[
            pltpu.VMEM((n_ch, _CH), jnp.int32),
            pltpu.VMEM((per_w, d), jnp.float32),
            pltpu.SemaphoreType.DMA,
        ],
    )
    def gk(table_hbm, idx_hbm, out_hbm, idx_v, rows_v, sem):
        wid = lax.axis_index("s") * _NC + lax.axis_index("c")
        base = wid * per_w
        pltpu.sync_copy(idx_hbm.at[pl.ds(wid * n_ch, n_ch)], idx_v)
        copies = []
        for j in range(n_ch):
            copies.append(
                pltpu.async_copy(
                    table_hbm.at[idx_v.at[j]],
                    rows_v.at[pl.ds(j * _CH, _CH)],
                    sem,
                )
            )
        for c in copies:
            c.wait()
        pltpu.sync_copy(rows_v, out_hbm.at[pl.ds(base, per_w)])

    return gk(table2d, idx2d)


def _tc_compose(g2, b, t_steps):
    blk = 512
    grid = b // blk

    def body(g_ref, c_ref, s_ref):
        r = lax.broadcasted_iota(jnp.int32, (32, 32), 0)
        j = lax.broadcasted_iota(jnp.int32, (32, 32), 1)
        P = jnp.where(j == 4 * (r % 8) + r // 8, 1.0, 0.0).astype(jnp.float32)
        dn_ext = (((1,), (1,)), ((), ()))
        dn_sto = (((0,), (0,)), ((), ()))
        hi = lax.Precision.HIGHEST

        Cw = jnp.ones((8, blk), jnp.float32)
        Cx = jnp.zeros((8, blk), jnp.float32)
        Cy = jnp.zeros((8, blk), jnp.float32)
        Cz = jnp.zeros((8, blk), jnp.float32)
        for t in range(t_steps):
            gt = g_ref[:, t * 32:(t + 1) * 32]
            R = lax.dot_general(P, gt, dn_ext, precision=hi)
            Gw, Gx, Gy, Gz = R[0:8], R[8:16], R[16:24], R[24:32]
            n2 = Gw * Gw + Gx * Gx + Gy * Gy + Gz * Gz
            inv = 1.0 / jnp.maximum(jnp.sqrt(n2), 1e-12)
            Gw, Gx, Gy, Gz = Gw * inv, Gx * inv, Gy * inv, Gz * inv
            w = Cw * Gw - Cx * Gx - Cy * Gy - Cz * Gz
            x = Cw * Gx + Cx * Gw + Cy * Gz - Cz * Gy
            y = Cw * Gy - Cx * Gz + Cy * Gw + Cz * Gx
            z = Cw * Gz + Cx * Gy - Cy * Gx + Cz * Gw
            n2c = w * w + x * x + y * y + z * z
            invc = 1.0 / jnp.maximum(jnp.sqrt(n2c), 1e-12)
            Cw, Cx, Cy, Cz = w * invc, x * invc, y * invc, z * invc
            a = jnp.minimum(jnp.abs(Cw), 1.0 - 1e-7)
            pf = jnp.sqrt(1.0 - a) * (
                1.5707288
                + a * (-0.2121144 + a * (0.0742610 + a * (-0.0187293)))
            )
            s_ref[:, t] = jnp.mean(pf, axis=0)
        Cstack = jnp.concatenate([Cw, Cx, Cy, Cz], axis=0)
        c_ref[...] = lax.dot_general(Cstack, P, dn_sto, precision=hi)

    return pl.pallas_call(
        body,
        grid=(grid,),
        in_specs=[pl.BlockSpec((blk, t_steps * 32), lambda i: (i, 0))],
        out_specs=[
            pl.BlockSpec((blk, 32), lambda i: (i, 0)),
            pl.BlockSpec((blk, t_steps), lambda i: (i, 0)),
        ],
        out_shape=[
            jax.ShapeDtypeStruct((b, 32), jnp.float32),
            jax.ShapeDtypeStruct((b, t_steps), jnp.float32),
        ],
    )(g2)


def kernel(tokens, embed):
    b, t_steps = tokens.shape
    vocab, m, four = embed.shape
    d = m * four
    bt = b * t_steps
    table2d = embed.reshape(vocab, d)
    idx2d = tokens.astype(jnp.int32).reshape(bt // _CH, _CH)
    gathered = _sc_gather(table2d, idx2d, bt, d)
    g2 = gathered.reshape(b, t_steps * d)
    c2, sigmas = _tc_compose(g2, b, t_steps)
    return c2.reshape(b, m, four), sigmas


# speedup vs baseline: 3.5835x; 3.5835x over previous
import functools

import jax
import jax.numpy as jnp
from jax import lax
from jax.experimental import pallas as pl
from jax.experimental.pallas import tpu as pltpu
from jax.experimental.pallas import tpu_sc as plsc

_NC = 2
_NS = 16
_CH = 128


def _sc_gather(table2d, idx1d, bt, d):
    nw = _NC * _NS
    per_w = bt // nw
    n_ch = per_w // _CH
    mesh = plsc.VectorSubcoreMesh(core_axis_name="c", subcore_axis_name="s")

    @functools.partial(
        pl.kernel,
        mesh=mesh,
        out_type=jax.ShapeDtypeStruct((bt, d), jnp.float32),
        scratch_types=[
            pltpu.VMEM((per_w,), jnp.int32),
            pltpu.VMEM((per_w, d), jnp.float32),
            pltpu.SemaphoreType.DMA,
        ],
        compiler_params=pltpu.CompilerParams(use_tc_tiling_on_sc=False),
    )
    def gk(table_hbm, idx_hbm, out_hbm, idx_v, rows_v, sem):
        wid = lax.axis_index("s") * _NC + lax.axis_index("c")
        base = wid * per_w
        pltpu.sync_copy(idx_hbm.at[pl.ds(base, per_w)], idx_v)
        copies = []
        for j in range(n_ch):
            copies.append(
                pltpu.async_copy(
                    table_hbm.at[idx_v.at[pl.ds(j * _CH, _CH)]],
                    rows_v.at[pl.ds(j * _CH, _CH)],
                    sem,
                )
            )
        for c in copies:
            c.wait()
        pltpu.sync_copy(rows_v, out_hbm.at[pl.ds(base, per_w)])

    return gk(table2d, idx1d)


def _tc_compose(g2, b, t_steps):
    blk = 512
    grid = b // blk

    def body(g_ref, c_ref, s_ref):
        r = lax.broadcasted_iota(jnp.int32, (32, 32), 0)
        j = lax.broadcasted_iota(jnp.int32, (32, 32), 1)
        P = jnp.where(j == 4 * (r % 8) + r // 8, 1.0, 0.0).astype(jnp.float32)
        dn_ext = (((1,), (1,)), ((), ()))
        dn_sto = (((0,), (0,)), ((), ()))
        hi = lax.Precision.HIGHEST

        Cw = jnp.ones((8, blk), jnp.float32)
        Cx = jnp.zeros((8, blk), jnp.float32)
        Cy = jnp.zeros((8, blk), jnp.float32)
        Cz = jnp.zeros((8, blk), jnp.float32)
        for t in range(t_steps):
            gt = g_ref[:, t * 32:(t + 1) * 32]
            R = lax.dot_general(P, gt, dn_ext, precision=hi)
            Gw, Gx, Gy, Gz = R[0:8], R[8:16], R[16:24], R[24:32]
            n2 = Gw * Gw + Gx * Gx + Gy * Gy + Gz * Gz
            inv = 1.0 / jnp.maximum(jnp.sqrt(n2), 1e-12)
            Gw, Gx, Gy, Gz = Gw * inv, Gx * inv, Gy * inv, Gz * inv
            w = Cw * Gw - Cx * Gx - Cy * Gy - Cz * Gz
            x = Cw * Gx + Cx * Gw + Cy * Gz - Cz * Gy
            y = Cw * Gy - Cx * Gz + Cy * Gw + Cz * Gx
            z = Cw * Gz + Cx * Gy - Cy * Gx + Cz * Gw
            n2c = w * w + x * x + y * y + z * z
            invc = 1.0 / jnp.maximum(jnp.sqrt(n2c), 1e-12)
            Cw, Cx, Cy, Cz = w * invc, x * invc, y * invc, z * invc
            a = jnp.minimum(jnp.abs(Cw), 1.0 - 1e-7)
            pf = jnp.sqrt(1.0 - a) * (
                1.5707288
                + a * (-0.2121144 + a * (0.0742610 + a * (-0.0187293)))
            )
            s_ref[:, t] = jnp.mean(pf, axis=0)
        Cstack = jnp.concatenate([Cw, Cx, Cy, Cz], axis=0)
        c_ref[...] = lax.dot_general(Cstack, P, dn_sto, precision=hi)

    return pl.pallas_call(
        body,
        grid=(grid,),
        in_specs=[pl.BlockSpec((blk, t_steps * 32), lambda i: (i, 0))],
        out_specs=[
            pl.BlockSpec((blk, 32), lambda i: (i, 0)),
            pl.BlockSpec((blk, t_steps), lambda i: (i, 0)),
        ],
        out_shape=[
            jax.ShapeDtypeStruct((b, 32), jnp.float32),
            jax.ShapeDtypeStruct((b, t_steps), jnp.float32),
        ],
    )(g2)


def kernel(tokens, embed):
    b, t_steps = tokens.shape
    vocab, m, four = embed.shape
    d = m * four
    bt = b * t_steps
    table2d = embed.reshape(vocab, d)
    idx1d = tokens.astype(jnp.int32).reshape(bt)
    gathered = _sc_gather(table2d, idx1d, bt, d)
    g2 = gathered.reshape(b, t_steps * d)
    c2, sigmas = _tc_compose(g2, b, t_steps)
    return c2.reshape(b, m, four), sigmas
